# Initial kernel scaffold; baseline (speedup 1.0000x reference)
import jax
import jax.numpy as jnp
from jax.experimental import pallas as pl


def kernel(features, neighbors):
    raise NotImplementedError("write your pallas kernel here")


# speedup vs baseline: 1.1479x; 1.1479x over previous
import functools

import jax
import jax.numpy as jnp
from jax import lax
from jax.experimental import pallas as pl
from jax.experimental.pallas import tpu as pltpu
from jax.experimental.pallas import tpu_sc as plsc

N_PAD = 10240
NODES_PER_WORKER = 320
CHUNK = 4
K = 32
D = 128
NUM_CHUNKS = NODES_PER_WORKER // CHUNK
NUM_WORKERS = 32


def _sc_mean_aggregate(features, nbr_flat):
    mesh = plsc.VectorSubcoreMesh(core_axis_name="c", subcore_axis_name="s")

    @functools.partial(
        pl.kernel,
        mesh=mesh,
        out_type=jax.ShapeDtypeStruct((N_PAD, D), jnp.float32),
        scratch_types=[
            pltpu.VMEM((CHUNK * K,), jnp.int32),
            pltpu.VMEM((CHUNK * K, D), jnp.float32),
            pltpu.VMEM((CHUNK, D), jnp.float32),
            pltpu.SemaphoreType.DMA,
        ],
    )
    def kern(feat_hbm, nbr_hbm, out_hbm, idx_v, rows_v, out_v, sem):
        wid = lax.axis_index("s") * 2 + lax.axis_index("c")
        base = wid * NODES_PER_WORKER

        def chunk_body(j, carry):
            node0 = base + j * CHUNK
            pltpu.sync_copy(nbr_hbm.at[pl.ds(node0 * K, CHUNK * K)], idx_v)
            pltpu.async_copy(feat_hbm.at[idx_v], rows_v, sem).wait()
            for c in range(CHUNK):
                for v in range(D // 16):
                    sl = pl.ds(v * 16, 16)

                    def red(kk, acc, c=c, sl=sl):
                        return acc + rows_v[c * K + kk, sl]

                    acc = lax.fori_loop(0, K, red, jnp.zeros((16,), jnp.float32))
                    out_v[c, sl] = acc * (1.0 / K)
            pltpu.sync_copy(out_v, out_hbm.at[pl.ds(node0, CHUNK)])
            return carry

        lax.fori_loop(0, NUM_CHUNKS, chunk_body, 0)

    return kern(features, nbr_flat)


def kernel(features, neighbors):
    n = neighbors.shape[0]
    nbr = jnp.pad(neighbors.astype(jnp.int32), ((0, N_PAD - n), (0, 0)))
    out = _sc_mean_aggregate(features, nbr.reshape(-1))
    return out[:n]


# speedup vs baseline: 1.4951x; 1.3025x over previous
import functools

import jax
import jax.numpy as jnp
from jax import lax
from jax.experimental import pallas as pl
from jax.experimental.pallas import tpu as pltpu
from jax.experimental.pallas import tpu_sc as plsc

N_PAD = 10240
NODES_PER_WORKER = 320
CHUNK = 4
K = 32
D = 128
NUM_CHUNKS = NODES_PER_WORKER // CHUNK
HALF = NUM_CHUNKS // 2


def _tree_sum(terms):
    while len(terms) > 1:
        nxt = [terms[i] + terms[i + 1] for i in range(0, len(terms) - 1, 2)]
        if len(terms) % 2:
            nxt.append(terms[-1])
        terms = nxt
    return terms[0]


def _sc_mean_aggregate(features, nbr_flat):
    mesh = plsc.VectorSubcoreMesh(core_axis_name="c", subcore_axis_name="s")

    @functools.partial(
        pl.kernel,
        mesh=mesh,
        out_type=jax.ShapeDtypeStruct((N_PAD, D), jnp.float32),
        scratch_types=[
            pltpu.VMEM((NODES_PER_WORKER * K,), jnp.int32),
            pltpu.VMEM((CHUNK * K, D), jnp.float32),
            pltpu.VMEM((CHUNK * K, D), jnp.float32),
            pltpu.VMEM((CHUNK, D), jnp.float32),
            pltpu.SemaphoreType.DMA,
            pltpu.SemaphoreType.DMA,
        ],
    )
    def kern(feat_hbm, nbr_hbm, out_hbm, idx_all, rows0, rows1, out_v,
             sem0, sem1):
        wid = lax.axis_index("s") * 2 + lax.axis_index("c")
        base = wid * NODES_PER_WORKER
        pltpu.sync_copy(nbr_hbm.at[pl.ds(base * K, NODES_PER_WORKER * K)],
                        idx_all)

        def gsrc(j):
            return feat_hbm.at[idx_all.at[pl.ds(j * CHUNK * K, CHUNK * K)]]

        def fire(j, rows, sem):
            pltpu.async_copy(gsrc(j), rows, sem)

        def drain(j, rows, sem):
            pltpu.make_async_copy(gsrc(j), rows, sem).wait()

        def reduce(rows, j):
            for c in range(CHUNK):
                for v in range(D // 16):
                    sl = pl.ds(v * 16, 16)
                    acc = _tree_sum([rows[c * K + kk, sl] for kk in range(K)])
                    out_v[c, sl] = acc * (1.0 / K)
            pltpu.sync_copy(out_v, out_hbm.at[pl.ds(base + j * CHUNK, CHUNK)])

        fire(0, rows0, sem0)

        def body(jj, carry):
            j = jj * 2
            fire(j + 1, rows1, sem1)
            drain(j, rows0, sem0)
            reduce(rows0, j)

            @pl.when(jj < HALF - 1)
            def _():
                fire(j + 2, rows0, sem0)

            drain(j + 1, rows1, sem1)
            reduce(rows1, j + 1)
            return carry

        lax.fori_loop(0, HALF, body, 0)

    return kern(features, nbr_flat)


def kernel(features, neighbors):
    n = neighbors.shape[0]
    nbr = jnp.pad(neighbors.astype(jnp.int32), ((0, N_PAD - n), (0, 0)))
    out = _sc_mean_aggregate(features, nbr.reshape(-1))
    return out[:n]
